# baseline (speedup 1.0000x reference)
import functools

import jax
import jax.numpy as jnp
from jax import lax
from jax.experimental import pallas as pl
from jax.experimental.pallas import tpu as pltpu
from jax.experimental.pallas import tpu_sc as plsc

G = 512
C = 128
W = 128
F32 = jnp.float32
BF16 = jnp.bfloat16
CB = 500
NSC = 32


def _pick_blk(n):
    for b in (10000, 4000, 2000, 1600, 1000, 800, 400, 200, 100, 50, 40, 25,
              20, 10, 8, 5, 4, 2, 1):
        if n % b == 0 and (b % 8 == 0 or b == n):
            return b
    return n


def _score_body(nb, x_op, x_ma, x_ag, b_op, b_ma, b_ag,
                tok_op, wl_op, bl_op, wr_op, br_op, att_op,
                tok_ma, wl_ma, bl_ma, wr_ma, br_ma, att_ma,
                tok_ag, wl_ag, bl_ag, wr_ag, br_ag, att_ag,
                w_op, w_ma, w_ag, d_op, d_ma, d_ag,
                den_op, den_ma, den_ag):
    i = pl.program_id(0)
    types = (
        (x_op, b_op, tok_op, wl_op, bl_op, wr_op, br_op, att_op, w_op, den_op),
        (x_ma, b_ma, tok_ma, wl_ma, bl_ma, wr_ma, br_ma, att_ma, w_ma, den_ma),
        (x_ag, b_ag, tok_ag, wl_ag, bl_ag, wr_ag, br_ag, att_ag, w_ag, den_ag),
    )

    @pl.when(i == 0)
    def _init():
        for (*_, den) in types:
            den[...] = jnp.zeros_like(den)

    for (x, b, tok, wl, bl, wr, br, att, w_out, den) in types:
        xb = x[...]
        blk = xb.shape[0]
        gl = jax.lax.dot_general(xb, wl[...], (((1,), (1,)), ((), ())),
                                 preferred_element_type=F32) + bl[...]
        gr = jax.lax.dot_general(tok[...], wr[...], (((1,), (1,)), ((), ())),
                                 preferred_element_type=F32) + br[...]
        e = gl + gr
        e = jnp.where(e >= 0, e, 0.2 * e)
        score = jax.lax.dot_general(att[...], e, (((1,), (1,)), ((), ())),
                                    preferred_element_type=F32)
        wrow = jnp.exp(score)
        w_out[...] = wrow.reshape(1, 1, blk)

        bb = b[...].reshape(1, blk)
        bfirst = jnp.min(bb)
        blast = jnp.max(bb)
        base = jnp.minimum((bfirst // 8) * 8, G - W)
        span_ok = (blast - base) < W
        ones8 = jnp.ones((blk, 8), BF16)

        @pl.when(span_ok)
        def _narrow():
            rel = bb - base
            ohw = jnp.where(jax.lax.broadcasted_iota(jnp.int32, (W, blk), 0)
                            == rel, wrow, 0.0).astype(BF16)
            dden = jax.lax.dot_general(ohw, ones8, (((1,), (0,)), ((), ())),
                                       preferred_element_type=F32)
            den[pl.ds(pl.multiple_of(base, 8), W), :] += dden[:, :1]

        @pl.when(jnp.logical_not(span_ok))
        def _full():
            seg = jax.lax.broadcasted_iota(jnp.int32, (G, blk), 0)
            oh = jnp.where(seg == bb, wrow, 0.0).astype(BF16)
            dden = jax.lax.dot_general(oh, ones8, (((1,), (0,)), ((), ())),
                                       preferred_element_type=F32)
            den[...] += dden[:, :1]

    @pl.when(i == nb - 1)
    def _fin():
        d_op[...] = den_op[...]
        d_ma[...] = den_ma[...]
        d_ag[...] = den_ag[...]


def _sc_segsum(x3, w2, b3):
    nch = w2.shape[0]
    jmax = (nch + NSC - 1) // NSC
    nsub = CB // 125
    mesh = plsc.VectorSubcoreMesh(core_axis_name="c", subcore_axis_name="s")

    @functools.partial(
        pl.kernel, mesh=mesh,
        out_type=jax.ShapeDtypeStruct((2, G, C), F32),
        scratch_types=[
            pltpu.VMEM((CB, C), F32),
            pltpu.VMEM((CB,), F32),
            pltpu.VMEM((nsub, 125), jnp.int32),
            pltpu.VMEM_SHARED((G, C), F32),
        ],
    )
    def k(x_hbm, w_hbm, b_hbm, out_hbm, xv, wv, bv, accs):
        cid = lax.axis_index("c")
        sid = lax.axis_index("s")
        wid = sid * 2 + cid

        def zrow(r, _):
            for t in range(C // 16):
                xv[r, pl.ds(t * 16, 16)] = jnp.zeros((16,), F32)
            return 0
        lax.fori_loop(0, 16, zrow, 0)
        s0 = pl.multiple_of(sid * 32, 8)
        s1 = pl.multiple_of(sid * 32 + 16, 8)
        pltpu.sync_copy(xv.at[pl.ds(0, 16)], accs.at[pl.ds(s0, 16)])
        pltpu.sync_copy(xv.at[pl.ds(0, 16)], accs.at[pl.ds(s1, 16)])
        plsc.subcore_barrier()

        def chunk(j, _):
            ch = j * NSC + wid

            @pl.when(ch < nch)
            def _do():
                pltpu.sync_copy(x_hbm.at[ch], xv)
                pltpu.sync_copy(w_hbm.at[ch], wv)
                pltpu.sync_copy(b_hbm.at[ch], bv)

                def grp(g, _):
                    br = g * 16
                    wgrp = wv[pl.ds(br, 16)]
                    for k2 in range(16):
                        wr = wgrp[k2]
                        for t in range(C // 16):
                            sl = pl.ds(t * 16, 16)
                            xv[br + k2, sl] = xv[br + k2, sl] * wr
                    return 0
                lax.fori_loop(0, CB // 16, grp, 0)
                rem = CB % 16
                if rem:
                    wlast = wv[pl.ds(CB - 16, 16)]
                    for k2 in range(16 - rem, 16):
                        wr = wlast[k2]
                        r = CB - 16 + k2
                        for t in range(C // 16):
                            sl = pl.ds(t * 16, 16)
                            xv[r, sl] = xv[r, sl] * wr
                for q in range(nsub):
                    pltpu.sync_copy(xv.at[pl.ds(q * 125, 125)],
                                    accs.at[bv.at[q]], add=True)
            return 0
        lax.fori_loop(0, jmax, chunk, 0)
        plsc.subcore_barrier()

        pltpu.sync_copy(accs.at[pl.ds(s0, 16)],
                        out_hbm.at[cid, pl.ds(s0, 16)])
        pltpu.sync_copy(accs.at[pl.ds(s1, 16)],
                        out_hbm.at[cid, pl.ds(s1, 16)])

    return k(x3, w2, b3)


def _fin_body(a_op, a_ma, a_ag, d_op, d_ma, d_ag,
              wl_op, bl_op, bias_op, wl_ma, bl_ma, bias_ma,
              wl_ag, bl_ag, bias_ag, mw1, mb1, mw2, mb2,
              o_op, o_ma, o_ag, o_gf):
    gs = []
    for acc, dref, wl, bl, bias, out in (
            (a_op, d_op, wl_op, bl_op, bias_op, o_op),
            (a_ma, d_ma, wl_ma, bl_ma, bias_ma, o_ma),
            (a_ag, d_ag, wl_ag, bl_ag, bias_ag, o_ag)):
        den = dref[...]
        sx = (acc[0] + acc[1]) / jnp.maximum(den, 1e-16)
        mask = jnp.where(den > 0, 1.0, 0.0)
        g = jax.lax.dot_general(sx, wl[...], (((1,), (1,)), ((), ())),
                                preferred_element_type=F32)
        g = g + mask * bl[...] + bias[...]
        out[...] = g
        gs.append(g)
    h = jnp.concatenate(gs, axis=1)
    h = jax.lax.dot_general(h, mw1[...], (((1,), (1,)), ((), ())),
                            preferred_element_type=F32) + mb1[...]
    h = jnp.where(h >= 0, h, 0.01 * h)
    gf = jax.lax.dot_general(h, mw2[...], (((1,), (1,)), ((), ())),
                             preferred_element_type=F32) + mb2[...]
    o_gf[...] = gf


def kernel(x_operation, x_machine, x_AGV, batch_operation, batch_machine, batch_AGV,
           token_operation, Wl_operation, bl_operation, Wr_operation, br_operation,
           att_operation, bias_operation,
           token_machine, Wl_machine, bl_machine, Wr_machine, br_machine,
           att_machine, bias_machine,
           token_AGV, Wl_AGV, bl_AGV, Wr_AGV, br_AGV, att_AGV, bias_AGV,
           mix_W1, mix_b1, mix_W2, mix_b2):
    n = x_operation.shape[0]
    blk = _pick_blk(n)
    nb = n // blk
    gg = mix_W1.shape[0]

    row2 = lambda v: v.reshape(1, -1)
    b3 = lambda b: b.reshape(nb, 1, blk)

    x_spec = pl.BlockSpec((blk, C), lambda i: (i, 0))
    b_spec = pl.BlockSpec((1, 1, blk), lambda i: (i, 0, 0))
    full2 = lambda a: pl.BlockSpec(a.shape, lambda i: (0, 0))

    params = []
    specs = [x_spec, x_spec, x_spec, b_spec, b_spec, b_spec]
    for tok, wl, bl, wr, br, att in (
            (token_operation, Wl_operation, bl_operation, Wr_operation,
             br_operation, att_operation),
            (token_machine, Wl_machine, bl_machine, Wr_machine, br_machine,
             att_machine),
            (token_AGV, Wl_AGV, bl_AGV, Wr_AGV, br_AGV, att_AGV)):
        args = (row2(tok), wl, row2(bl), wr, row2(br), row2(att))
        params.extend(args)
        specs.extend(full2(a) for a in args)

    w_shape = jax.ShapeDtypeStruct((nb, 1, blk), F32)
    d_shape = jax.ShapeDtypeStruct((G, 1), F32)
    w_spec = pl.BlockSpec((1, 1, blk), lambda i: (i, 0, 0))
    d_spec = pl.BlockSpec((G, 1), lambda i: (0, 0))

    ws_and_dens = pl.pallas_call(
        functools.partial(_score_body, nb),
        grid=(nb,),
        in_specs=specs,
        out_specs=(w_spec,) * 3 + (d_spec,) * 3,
        out_shape=(w_shape,) * 3 + (d_shape,) * 3,
        scratch_shapes=[pltpu.VMEM((G, 1), F32)] * 3,
        compiler_params=pltpu.CompilerParams(
            dimension_semantics=("arbitrary",),
        ),
    )(x_operation, x_machine, x_AGV,
      b3(batch_operation), b3(batch_machine), b3(batch_AGV), *params)
    w_op, w_ma, w_ag, d_op, d_ma, d_ag = ws_and_dens

    nch = n // CB
    accs = []
    for x, wv, bv in ((x_operation, w_op, batch_operation),
                      (x_machine, w_ma, batch_machine),
                      (x_AGV, w_ag, batch_AGV)):
        accs.append(_sc_segsum(x.reshape(nch, CB, C), wv.reshape(nch, CB),
                               bv.reshape(nch, CB // 125, 125)))

    fin_in = (accs[0], accs[1], accs[2], d_op, d_ma, d_ag,
              Wl_operation, row2(bl_operation), row2(bias_operation),
              Wl_machine, row2(bl_machine), row2(bias_machine),
              Wl_AGV, row2(bl_AGV), row2(bias_AGV),
              mix_W1, row2(mix_b1), mix_W2, row2(mix_b2))
    fin_specs = [pl.BlockSpec(a.shape, (lambda nd: lambda i: (0,) * nd)(a.ndim))
                 for a in fin_in]
    out_shape = (
        jax.ShapeDtypeStruct((G, C), F32),
        jax.ShapeDtypeStruct((G, C), F32),
        jax.ShapeDtypeStruct((G, C), F32),
        jax.ShapeDtypeStruct((G, gg), F32),
    )
    out_specs = tuple(pl.BlockSpec(s.shape, lambda i: (0, 0)) for s in out_shape)
    return pl.pallas_call(
        _fin_body,
        grid=(1,),
        in_specs=fin_specs,
        out_specs=out_specs,
        out_shape=out_shape,
    )(*fin_in)


# speedup vs baseline: 4.1733x; 4.1733x over previous
import functools

import jax
import jax.numpy as jnp
from jax.experimental import pallas as pl
from jax.experimental.pallas import tpu as pltpu

G = 512
C = 128
W = 128
F32 = jnp.float32
BF16 = jnp.bfloat16


def _pick_blk(n):
    for b in (10000, 4000, 2000, 1600, 1000, 800, 400, 200, 100, 50, 40, 25, 20, 10, 8, 5, 4, 2, 1):
        if n % b == 0 and (b % 8 == 0 or b == n):
            return b
    return n


def _body(nb, x_op, x_ma, x_ag, b_op, b_ma, b_ag,
          tok_op, wl_op, bl_op, wr_op, br_op, att_op, bias_op,
          tok_ma, wl_ma, bl_ma, wr_ma, br_ma, att_ma, bias_ma,
          tok_ag, wl_ag, bl_ag, wr_ag, br_ag, att_ag, bias_ag,
          mw1, mb1, mw2, mb2,
          o_op, o_ma, o_ag, o_gf,
          num_op, den_op, num_ma, den_ma, num_ag, den_ag):
    i = pl.program_id(0)
    types = (
        (x_op, b_op, tok_op, wl_op, bl_op, wr_op, br_op, att_op, num_op, den_op),
        (x_ma, b_ma, tok_ma, wl_ma, bl_ma, wr_ma, br_ma, att_ma, num_ma, den_ma),
        (x_ag, b_ag, tok_ag, wl_ag, bl_ag, wr_ag, br_ag, att_ag, num_ag, den_ag),
    )

    @pl.when(i == 0)
    def _init():
        for (_, _, _, _, _, _, _, _, num, den) in types:
            num[...] = jnp.zeros_like(num)
            den[...] = jnp.zeros_like(den)

    for (x, b, tok, wl, bl, wr, br, att, num, den) in types:
        xb = x[...]
        blk = xb.shape[0]
        gl = jax.lax.dot_general(xb, wl[...], (((1,), (1,)), ((), ())),
                                 preferred_element_type=F32) + bl[...]
        gr = jax.lax.dot_general(tok[...], wr[...], (((1,), (1,)), ((), ())),
                                 preferred_element_type=F32) + br[...]
        e = gl + gr
        e = jnp.where(e >= 0, e, 0.2 * e)
        score = jax.lax.dot_general(att[...], e, (((1,), (1,)), ((), ())),
                                    preferred_element_type=F32)
        wrow = jnp.exp(score)
        glc = jnp.concatenate([gl.astype(BF16),
                               jnp.ones((blk, 8), BF16)], axis=1)
        bb = b[...].reshape(1, blk)
        bfirst = jnp.min(bb)
        blast = jnp.max(bb)
        base = jnp.minimum((bfirst // 8) * 8, G - W)
        span_ok = (blast - base) < W

        @pl.when(span_ok)
        def _narrow():
            rel = bb - base
            ohw = jnp.where(jax.lax.broadcasted_iota(jnp.int32, (W, blk), 0)
                            == rel, wrow, 0.0).astype(BF16)
            dcomb = jax.lax.dot_general(ohw, glc, (((1,), (0,)), ((), ())),
                                        preferred_element_type=F32)
            sl = pl.ds(pl.multiple_of(base, 8), W)
            num[sl, :] += dcomb[:, :C]
            den[sl, :] += dcomb[:, C:C + 1]

        @pl.when(jnp.logical_not(span_ok))
        def _full():
            seg = jax.lax.broadcasted_iota(jnp.int32, (G, blk), 0)
            oh = jnp.where(seg == bb, wrow, 0.0).astype(BF16)
            dcomb = jax.lax.dot_general(oh, glc, (((1,), (0,)), ((), ())),
                                        preferred_element_type=F32)
            num[...] += dcomb[:, :C]
            den[...] += dcomb[:, C:C + 1]

    @pl.when(i == nb - 1)
    def _finish():
        gs = []
        for (_, _, _, _, _, _, _, _, num, den), bias, out in (
                (types[0], bias_op, o_op), (types[1], bias_ma, o_ma),
                (types[2], bias_ag, o_ag)):
            g = num[...] / jnp.maximum(den[...], 1e-16) + bias[...]
            out[...] = g
            gs.append(g)
        h = jnp.concatenate(gs, axis=1)
        h = jax.lax.dot_general(h, mw1[...], (((1,), (1,)), ((), ())),
                                preferred_element_type=F32) + mb1[...]
        h = jnp.where(h >= 0, h, 0.01 * h)
        gf = jax.lax.dot_general(h, mw2[...], (((1,), (1,)), ((), ())),
                                 preferred_element_type=F32) + mb2[...]
        o_gf[...] = gf


def kernel(x_operation, x_machine, x_AGV, batch_operation, batch_machine, batch_AGV,
           token_operation, Wl_operation, bl_operation, Wr_operation, br_operation,
           att_operation, bias_operation,
           token_machine, Wl_machine, bl_machine, Wr_machine, br_machine,
           att_machine, bias_machine,
           token_AGV, Wl_AGV, bl_AGV, Wr_AGV, br_AGV, att_AGV, bias_AGV,
           mix_W1, mix_b1, mix_W2, mix_b2):
    n = x_operation.shape[0]
    blk = _pick_blk(n)
    nb = n // blk
    gg = mix_W1.shape[0]

    row2 = lambda v: v.reshape(1, -1)
    col2 = lambda v: v.reshape(-1, 1)
    b3 = lambda b: b.reshape(nb, 1, blk)

    x_spec = pl.BlockSpec((blk, C), lambda i: (i, 0))
    b_spec = pl.BlockSpec((1, 1, blk), lambda i: (i, 0, 0))
    full2 = lambda a: pl.BlockSpec(a.shape, lambda i: (0, 0))

    params = []
    specs = [x_spec, x_spec, x_spec, b_spec, b_spec, b_spec]
    for tok, wl, bl, wr, br, att, bias in (
            (token_operation, Wl_operation, bl_operation, Wr_operation, br_operation,
             att_operation, bias_operation),
            (token_machine, Wl_machine, bl_machine, Wr_machine, br_machine,
             att_machine, bias_machine),
            (token_AGV, Wl_AGV, bl_AGV, Wr_AGV, br_AGV, att_AGV, bias_AGV)):
        args = (row2(tok), wl, row2(bl), wr, row2(br), row2(att), row2(bias))
        params.extend(args)
        specs.extend(full2(a) for a in args)
    mix = (mix_W1, row2(mix_b1), mix_W2, row2(mix_b2))
    params.extend(mix)
    specs.extend(full2(a) for a in mix)

    out_shape = (
        jax.ShapeDtypeStruct((G, C), F32),
        jax.ShapeDtypeStruct((G, C), F32),
        jax.ShapeDtypeStruct((G, C), F32),
        jax.ShapeDtypeStruct((G, gg), F32),
    )
    out_specs = (
        pl.BlockSpec((G, C), lambda i: (0, 0)),
        pl.BlockSpec((G, C), lambda i: (0, 0)),
        pl.BlockSpec((G, C), lambda i: (0, 0)),
        pl.BlockSpec((G, gg), lambda i: (0, 0)),
    )
    scratch = []
    for _ in range(3):
        scratch.append(pltpu.VMEM((G, C), F32))
        scratch.append(pltpu.VMEM((G, 1), F32))

    return pl.pallas_call(
        functools.partial(_body, nb),
        grid=(nb,),
        in_specs=specs,
        out_specs=out_specs,
        out_shape=out_shape,
        scratch_shapes=scratch,
        compiler_params=pltpu.CompilerParams(
            dimension_semantics=("arbitrary",),
        ),
    )(x_operation, x_machine, x_AGV,
      b3(batch_operation), b3(batch_machine), b3(batch_AGV), *params)
